# initial kernel scaffold (unmeasured)
import jax
import jax.numpy as jnp
from jax import lax
from jax.experimental import pallas as pl
from jax.experimental.pallas import tpu as pltpu

N_DEV = 8
B, S, H, Dh, Dr = 2, 512, 16, 128, 32
D = 2048
BS = B * S
HL = H // N_DEV
KC = HL * Dh
QRC = HL * Dr
SCALE = (Dh + Dr) ** -0.5

f32 = jnp.float32
bf16 = jnp.bfloat16


def _body(x_ref, wdkv_ref, wuk_ref, wuv_ref, wq_ref, wqr_ref, wkr_ref,
          wo_ref, out_ref,
          kv_ref, rs_snd, rs_rcv, ag_snd, ag_rcv,
          rs_ssem, rs_rsem, ag_ssem, ag_rsem):
    my = lax.axis_index("i")
    right = lax.rem(my + 1, N_DEV)

    x = x_ref[...]

    c = jnp.dot(x, wdkv_ref[...], preferred_element_type=f32).astype(bf16)
    kp = jnp.dot(c, wuk_ref[...], preferred_element_type=f32).astype(bf16)
    vp = jnp.dot(c, wuv_ref[...], preferred_element_type=f32).astype(bf16)
    for j in range(N_DEV):
        kv_ref[j, :, :KC] = kp[:, j * KC:(j + 1) * KC]
        kv_ref[j, :, KC:] = vp[:, j * KC:(j + 1) * KC]

    def kv_chunk(i):
        return pl.load(kv_ref, (pl.ds(i, 1), slice(None), slice(None)))[0]

    s0 = lax.rem(my + N_DEV - 1, N_DEV)
    rs_snd[...] = kv_chunk(s0)
    acc = None
    for t in range(N_DEV - 1):
        rdma = pltpu.make_async_remote_copy(
            src_ref=rs_snd,
            dst_ref=rs_rcv.at[t],
            send_sem=rs_ssem.at[t],
            recv_sem=rs_rsem.at[t],
            device_id=(right,),
            device_id_type=pl.DeviceIdType.MESH,
        )
        rdma.start()
        rdma.wait()
        r = lax.rem(my + (2 * N_DEV - 2 - t), N_DEV)
        acc = rs_rcv[t, :, :].astype(f32) + kv_chunk(r).astype(f32)
        if t < N_DEV - 2:
            rs_snd[...] = acc.astype(bf16)
    k_mine = acc[:, :KC].astype(bf16)
    v_mine = acc[:, KC:].astype(bf16)

    q = jnp.dot(x, wq_ref[...], preferred_element_type=f32).astype(bf16)
    qr = jnp.dot(x, wqr_ref[...], preferred_element_type=f32).astype(bf16)
    kr = jnp.dot(x, wkr_ref[...], preferred_element_type=f32).astype(bf16)

    row_blocks = []
    for b in range(B):
        rows = slice(b * S, (b + 1) * S)
        krb = kr[rows]
        head_blocks = []
        for j in range(HL):
            cols = slice(j * Dh, (j + 1) * Dh)
            qh = q[rows, cols]
            kh = k_mine[rows, cols]
            qrh = qr[rows, j * Dr:(j + 1) * Dr]
            sc = lax.dot_general(qh, kh, (((1,), (1,)), ((), ())),
                                 preferred_element_type=f32)
            sc = sc + lax.dot_general(qrh, krb, (((1,), (1,)), ((), ())),
                                      preferred_element_type=f32)
            sc = sc * SCALE
            m = jnp.max(sc, axis=-1, keepdims=True)
            e = jnp.exp(sc - m)
            p = (e / jnp.sum(e, axis=-1, keepdims=True)).astype(bf16)
            head_blocks.append(
                jnp.dot(p, v_mine[rows, cols], preferred_element_type=f32))
        row_blocks.append(jnp.concatenate(head_blocks, axis=1))
    o_loc = jnp.concatenate(row_blocks, axis=0).astype(bf16)

    def wo_rows(i):
        return pl.load(wo_ref, (pl.ds(i * KC, KC), slice(None)))

    ag_snd[...] = o_loc
    oacc = jnp.dot(o_loc, wo_rows(my), preferred_element_type=f32)
    for h in range(N_DEV - 1):
        rdma = pltpu.make_async_remote_copy(
            src_ref=ag_snd,
            dst_ref=ag_rcv.at[h],
            send_sem=ag_ssem.at[h],
            recv_sem=ag_rsem.at[h],
            device_id=(right,),
            device_id_type=pl.DeviceIdType.MESH,
        )
        rdma.start()
        rdma.wait()
        ochunk = ag_rcv[h, :, :]
        r = lax.rem(my + (2 * N_DEV - 1 - h), N_DEV)
        oacc = oacc + jnp.dot(ochunk, wo_rows(r), preferred_element_type=f32)
        if h < N_DEV - 2:
            ag_snd[...] = ochunk

    out_ref[...] = oacc.reshape(B, S, D)


def kernel(x, Wdkv, Wuk, Wuv, Wq, Wqr, Wkr, Wo):
    idx = lax.axis_index("i")
    xf = x.reshape(BS, D).astype(bf16)
    wq_loc = lax.dynamic_slice(Wq, (0, idx * KC), (D, KC)).astype(bf16)
    wqr_loc = lax.dynamic_slice(Wqr, (0, idx * QRC), (D, QRC)).astype(bf16)

    return pl.pallas_call(
        _body,
        out_shape=jax.ShapeDtypeStruct((B, S, D), jnp.float32),
        in_specs=[pl.BlockSpec(memory_space=pltpu.VMEM)] * 8,
        out_specs=pl.BlockSpec(memory_space=pltpu.VMEM),
        scratch_shapes=[
            pltpu.VMEM((N_DEV, BS, 2 * KC), bf16),
            pltpu.VMEM((BS, 2 * KC), bf16),
            pltpu.VMEM((N_DEV - 1, BS, 2 * KC), bf16),
            pltpu.VMEM((BS, KC), bf16),
            pltpu.VMEM((N_DEV - 1, BS, KC), bf16),
            pltpu.SemaphoreType.DMA((N_DEV - 1,)),
            pltpu.SemaphoreType.DMA((N_DEV - 1,)),
            pltpu.SemaphoreType.DMA((N_DEV - 1,)),
            pltpu.SemaphoreType.DMA((N_DEV - 1,)),
        ],
        compiler_params=pltpu.CompilerParams(
            vmem_limit_bytes=112 * 1024 * 1024,
        ),
    )(xf, Wdkv.astype(bf16), Wuk.astype(bf16), Wuv.astype(bf16),
      wq_loc, wqr_loc, Wkr.astype(bf16), Wo.astype(bf16))


# baseline (device time: 218209 ns/iter reference)
import jax
import jax.numpy as jnp
from jax import lax
from jax.experimental import pallas as pl
from jax.experimental.pallas import tpu as pltpu

N_DEV = 8
B, S, H, Dh, Dr = 2, 512, 16, 128, 32
D = 2048
BS = B * S
HL = H // N_DEV
KC = HL * Dh
QRC = HL * Dr
SCALE = (Dh + Dr) ** -0.5

f32 = jnp.float32
bf16 = jnp.bfloat16


def _body(x_ref, wdkv_ref, wuk_ref, wuv_ref, wq_ref, wqr_ref, wkr_ref,
          wo_ref, out_ref, rs_snd, rs_rcv, ag_snd, ag_rcv,
          rs_ssem, rs_rsem, ag_ssem, ag_rsem):
    my = lax.axis_index("i")
    right = lax.rem(my + 1, N_DEV)

    c = jnp.dot(x_ref[...], wdkv_ref[...],
                preferred_element_type=f32).astype(bf16)

    def kv_chunk(i):
        k = jnp.dot(c, wuk_ref[:, pl.ds(i * KC, KC)],
                    preferred_element_type=f32)
        v = jnp.dot(c, wuv_ref[:, pl.ds(i * KC, KC)],
                    preferred_element_type=f32)
        return jnp.concatenate([k, v], axis=1)

    rs_snd[...] = kv_chunk(lax.rem(my + N_DEV - 1, N_DEV)).astype(bf16)
    acc = None
    for t in range(N_DEV - 1):
        rdma = pltpu.make_async_remote_copy(
            src_ref=rs_snd,
            dst_ref=rs_rcv.at[t],
            send_sem=rs_ssem.at[t],
            recv_sem=rs_rsem.at[t],
            device_id=(right,),
            device_id_type=pl.DeviceIdType.MESH,
        )
        rdma.start()
        rdma.wait()
        r = lax.rem(my + (2 * N_DEV - 2 - t), N_DEV)
        acc = rs_rcv[t, :, :].astype(f32) + kv_chunk(r)
        if t < N_DEV - 2:
            rs_snd[...] = acc.astype(bf16)
    k_mine = acc[:, :KC].astype(bf16)
    v_mine = acc[:, KC:].astype(bf16)

    q = jnp.dot(x_ref[...], wq_ref[...],
                preferred_element_type=f32).astype(bf16)
    qr = jnp.dot(x_ref[...], wqr_ref[...],
                 preferred_element_type=f32).astype(bf16)
    kr = jnp.dot(x_ref[...], wkr_ref[...],
                 preferred_element_type=f32).astype(bf16)

    for b in range(B):
        rows = slice(b * S, (b + 1) * S)
        krb = kr[rows]
        for j in range(HL):
            cols = slice(j * Dh, (j + 1) * Dh)
            qh = q[rows, cols]
            kh = k_mine[rows, cols]
            qrh = qr[rows, j * Dr:(j + 1) * Dr]
            sc = lax.dot_general(qh, kh, (((1,), (1,)), ((), ())),
                                 preferred_element_type=f32)
            sc = sc + lax.dot_general(qrh, krb, (((1,), (1,)), ((), ())),
                                      preferred_element_type=f32)
            sc = sc * SCALE
            m = jnp.max(sc, axis=-1, keepdims=True)
            e = jnp.exp(sc - m)
            p = (e / jnp.sum(e, axis=-1, keepdims=True)).astype(bf16)
            ag_snd[rows, cols] = jnp.dot(
                p, v_mine[rows, cols], preferred_element_type=f32
            ).astype(bf16)

    def wo_rows(i):
        return wo_ref[pl.ds(i * KC, KC), :]

    out_ref[...] = jnp.dot(ag_snd[...], wo_rows(my),
                           preferred_element_type=f32)
    for h in range(N_DEV - 1):
        rdma = pltpu.make_async_remote_copy(
            src_ref=ag_snd,
            dst_ref=ag_rcv.at[h],
            send_sem=ag_ssem.at[h],
            recv_sem=ag_rsem.at[h],
            device_id=(right,),
            device_id_type=pl.DeviceIdType.MESH,
        )
        rdma.start()
        rdma.wait()
        r = lax.rem(my + (2 * N_DEV - 1 - h), N_DEV)
        out_ref[...] = out_ref[...] + jnp.dot(
            ag_rcv[h, :, :], wo_rows(r), preferred_element_type=f32)
        if h < N_DEV - 2:
            ag_snd[...] = ag_rcv[h, :, :]


def kernel(x, Wdkv, Wuk, Wuv, Wq, Wqr, Wkr, Wo):
    idx = lax.axis_index("i")
    xf = x.reshape(BS, D).astype(bf16)
    wq_loc = lax.dynamic_slice(Wq, (0, idx * KC), (D, KC)).astype(bf16)
    wqr_loc = lax.dynamic_slice(Wqr, (0, idx * QRC), (D, QRC)).astype(bf16)

    out = pl.pallas_call(
        _body,
        out_shape=jax.ShapeDtypeStruct((BS, D), jnp.float32),
        in_specs=[pl.BlockSpec(memory_space=pltpu.VMEM)] * 8,
        out_specs=pl.BlockSpec(memory_space=pltpu.VMEM),
        scratch_shapes=[
            pltpu.VMEM((BS, 2 * KC), bf16),
            pltpu.VMEM((N_DEV - 1, BS, 2 * KC), bf16),
            pltpu.VMEM((BS, KC), bf16),
            pltpu.VMEM((N_DEV - 1, BS, KC), bf16),
            pltpu.SemaphoreType.DMA((N_DEV - 1,)),
            pltpu.SemaphoreType.DMA((N_DEV - 1,)),
            pltpu.SemaphoreType.DMA((N_DEV - 1,)),
            pltpu.SemaphoreType.DMA((N_DEV - 1,)),
        ],
        compiler_params=pltpu.CompilerParams(
            vmem_limit_bytes=62 * 1024 * 1024,
        ),
    )(xf, Wdkv.astype(bf16), Wuk.astype(bf16), Wuv.astype(bf16),
      wq_loc, wqr_loc, Wkr.astype(bf16), Wo.astype(bf16))
    return out.reshape(B, S, D)


# device time: 201254 ns/iter; 1.0842x vs baseline; 1.0842x over previous
import jax
import jax.numpy as jnp
from jax import lax
from jax.experimental import pallas as pl
from jax.experimental.pallas import tpu as pltpu

N_DEV = 8
B, S, H, Dh, Dr = 2, 512, 16, 128, 32
D = 2048
BS = B * S
HL = H // N_DEV
KC = HL * Dh
QRC = HL * Dr
SCALE = (Dh + Dr) ** -0.5

f32 = jnp.float32
bf16 = jnp.bfloat16


def _body(x_ref, wdkv_ref, wuk_ref, wuv_ref, wq_ref, wqr_ref, wkr_ref,
          wo_ref, out_ref, rs_snd, rs_rcv, ag_snd, ag_rcv,
          rs_ssem, rs_rsem, ag_ssem, ag_rsem):
    my = lax.axis_index("i")
    right = lax.rem(my + 1, N_DEV)

    c = jnp.dot(x_ref[...], wdkv_ref[...],
                preferred_element_type=f32).astype(bf16)

    def kv_chunk(i):
        k = jnp.dot(c, wuk_ref[:, pl.ds(i * KC, KC)],
                    preferred_element_type=f32)
        v = jnp.dot(c, wuv_ref[:, pl.ds(i * KC, KC)],
                    preferred_element_type=f32)
        return jnp.concatenate([k, v], axis=1)

    rs_rdmas = [
        pltpu.make_async_remote_copy(
            src_ref=rs_snd,
            dst_ref=rs_rcv.at[t],
            send_sem=rs_ssem.at[t],
            recv_sem=rs_rsem.at[t],
            device_id=(right,),
            device_id_type=pl.DeviceIdType.MESH,
        )
        for t in range(N_DEV - 1)
    ]
    rs_snd[...] = kv_chunk(lax.rem(my + N_DEV - 1, N_DEV)).astype(bf16)
    rs_rdmas[0].start()

    q = jnp.dot(x_ref[...], wq_ref[...],
                preferred_element_type=f32).astype(bf16)
    qr = jnp.dot(x_ref[...], wqr_ref[...],
                 preferred_element_type=f32).astype(bf16)
    kr = jnp.dot(x_ref[...], wkr_ref[...],
                 preferred_element_type=f32).astype(bf16)

    part = kv_chunk(lax.rem(my + N_DEV - 2, N_DEV))
    acc = None
    for t in range(N_DEV - 1):
        rs_rdmas[t].wait()
        acc = rs_rcv[t, :, :].astype(f32) + part
        if t < N_DEV - 2:
            rs_snd[...] = acc.astype(bf16)
            rs_rdmas[t + 1].start()
            part = kv_chunk(lax.rem(my + (2 * N_DEV - 3 - t), N_DEV))
    k_mine = acc[:, :KC].astype(bf16)
    v_mine = acc[:, KC:].astype(bf16)

    for b in range(B):
        rows = slice(b * S, (b + 1) * S)
        krb = kr[rows]
        for j in range(HL):
            cols = slice(j * Dh, (j + 1) * Dh)
            qh = q[rows, cols]
            kh = k_mine[rows, cols]
            qrh = qr[rows, j * Dr:(j + 1) * Dr]
            sc = lax.dot_general(qh, kh, (((1,), (1,)), ((), ())),
                                 preferred_element_type=f32)
            sc = sc + lax.dot_general(qrh, krb, (((1,), (1,)), ((), ())),
                                      preferred_element_type=f32)
            sc = sc * SCALE
            m = jnp.max(sc, axis=-1, keepdims=True)
            e = jnp.exp(sc - m)
            p = (e / jnp.sum(e, axis=-1, keepdims=True)).astype(bf16)
            ag_snd[rows, cols] = jnp.dot(
                p, v_mine[rows, cols], preferred_element_type=f32
            ).astype(bf16)

    def wo_rows(i):
        return wo_ref[pl.ds(i * KC, KC), :]

    ag_rdmas = [
        pltpu.make_async_remote_copy(
            src_ref=ag_snd,
            dst_ref=ag_rcv.at[h],
            send_sem=ag_ssem.at[h],
            recv_sem=ag_rsem.at[h],
            device_id=(right,),
            device_id_type=pl.DeviceIdType.MESH,
        )
        for h in range(N_DEV - 1)
    ]
    ag_rdmas[0].start()
    out_ref[...] = jnp.dot(ag_snd[...], wo_rows(my),
                           preferred_element_type=f32)
    for h in range(N_DEV - 1):
        ag_rdmas[h].wait()
        if h < N_DEV - 2:
            ag_snd[...] = ag_rcv[h, :, :]
            ag_rdmas[h + 1].start()
        r = lax.rem(my + (2 * N_DEV - 1 - h), N_DEV)
        out_ref[...] = out_ref[...] + jnp.dot(
            ag_rcv[h, :, :], wo_rows(r), preferred_element_type=f32)


def kernel(x, Wdkv, Wuk, Wuv, Wq, Wqr, Wkr, Wo):
    idx = lax.axis_index("i")
    xf = x.reshape(BS, D).astype(bf16)
    wq_loc = lax.dynamic_slice(Wq, (0, idx * KC), (D, KC)).astype(bf16)
    wqr_loc = lax.dynamic_slice(Wqr, (0, idx * QRC), (D, QRC)).astype(bf16)

    out = pl.pallas_call(
        _body,
        out_shape=jax.ShapeDtypeStruct((BS, D), jnp.float32),
        in_specs=[pl.BlockSpec(memory_space=pltpu.VMEM)] * 8,
        out_specs=pl.BlockSpec(memory_space=pltpu.VMEM),
        scratch_shapes=[
            pltpu.VMEM((BS, 2 * KC), bf16),
            pltpu.VMEM((N_DEV - 1, BS, 2 * KC), bf16),
            pltpu.VMEM((BS, KC), bf16),
            pltpu.VMEM((N_DEV - 1, BS, KC), bf16),
            pltpu.SemaphoreType.DMA((N_DEV - 1,)),
            pltpu.SemaphoreType.DMA((N_DEV - 1,)),
            pltpu.SemaphoreType.DMA((N_DEV - 1,)),
            pltpu.SemaphoreType.DMA((N_DEV - 1,)),
        ],
        compiler_params=pltpu.CompilerParams(
            vmem_limit_bytes=62 * 1024 * 1024,
        ),
    )(xf, Wdkv.astype(bf16), Wuk.astype(bf16), Wuv.astype(bf16),
      wq_loc, wqr_loc, Wkr.astype(bf16), Wo.astype(bf16))
    return out.reshape(B, S, D)


# device time: 147988 ns/iter; 1.4745x vs baseline; 1.3599x over previous
import jax
import jax.numpy as jnp
from jax import lax
from jax.experimental import pallas as pl
from jax.experimental.pallas import tpu as pltpu

N_DEV = 8
B, S, H, Dh, Dr = 2, 512, 16, 128, 32
D = 2048
BS = B * S
HL = H // N_DEV
KC = HL * Dh
QRC = HL * Dr
SCALE = (Dh + Dr) ** -0.5

f32 = jnp.float32
bf16 = jnp.bfloat16


def _body(x_ref, wdkv_ref, wuk_ref, wuv_ref, wq_ref, wqr_ref, wkr_ref,
          wo_ref, out_ref,
          rs_snd_r, rs_snd_l, rs_rcv_r, rs_rcv_l,
          ag_snd_r, ag_snd_l, ag_rcv_r, ag_rcv_l,
          rs_ssem_r, rs_rsem_r, rs_ssem_l, rs_rsem_l,
          ag_ssem_r, ag_rsem_r, ag_ssem_l, ag_rsem_l):
    my = lax.axis_index("i")
    right = lax.rem(my + 1, N_DEV)
    left = lax.rem(my + N_DEV - 1, N_DEV)

    def pos(k):
        return lax.rem(my + k, N_DEV)

    c = jnp.dot(x_ref[...], wdkv_ref[...],
                preferred_element_type=f32).astype(bf16)

    def k_chunk(i):
        return jnp.dot(c, wuk_ref[:, pl.ds(i * KC, KC)],
                       preferred_element_type=f32)

    def v_chunk(i):
        return jnp.dot(c, wuv_ref[:, pl.ds(i * KC, KC)],
                       preferred_element_type=f32)

    def rdma(src, dst, ssem, rsem, dev):
        return pltpu.make_async_remote_copy(
            src_ref=src, dst_ref=dst, send_sem=ssem, recv_sem=rsem,
            device_id=(dev,), device_id_type=pl.DeviceIdType.MESH)

    rs_r = [rdma(rs_snd_r, rs_rcv_r.at[t], rs_ssem_r.at[t],
                 rs_rsem_r.at[t], right) for t in range(N_DEV - 1)]
    rs_l = [rdma(rs_snd_l, rs_rcv_l.at[t], rs_ssem_l.at[t],
                 rs_rsem_l.at[t], left) for t in range(N_DEV - 1)]

    rs_snd_r[...] = k_chunk(pos(N_DEV - 1)).astype(bf16)
    rs_snd_l[...] = v_chunk(pos(1)).astype(bf16)
    rs_r[0].start()
    rs_l[0].start()

    q = jnp.dot(x_ref[...], wq_ref[...],
                preferred_element_type=f32).astype(bf16)
    qr = jnp.dot(x_ref[...], wqr_ref[...],
                 preferred_element_type=f32).astype(bf16)
    kr = jnp.dot(x_ref[...], wkr_ref[...],
                 preferred_element_type=f32).astype(bf16)
    part_k = k_chunk(pos(N_DEV - 2))
    part_v = v_chunk(pos(2))

    acc_k = acc_v = None
    for t in range(N_DEV - 1):
        rs_r[t].wait()
        rs_l[t].wait()
        acc_k = rs_rcv_r[t, :, :].astype(f32) + part_k
        acc_v = rs_rcv_l[t, :, :].astype(f32) + part_v
        if t < N_DEV - 2:
            rs_snd_r[...] = acc_k.astype(bf16)
            rs_snd_l[...] = acc_v.astype(bf16)
            rs_r[t + 1].start()
            rs_l[t + 1].start()
            part_k = k_chunk(pos(2 * N_DEV - 3 - t))
            part_v = v_chunk(pos(3 + t))
    k_mine = acc_k.astype(bf16)
    v_mine = acc_v.astype(bf16)

    ag_snd = [ag_snd_r, ag_snd_l]
    for b in range(B):
        rows = slice(b * S, (b + 1) * S)
        krb = kr[rows]
        for j in range(HL):
            cols = slice(j * Dh, (j + 1) * Dh)
            qh = q[rows, cols]
            kh = k_mine[rows, cols]
            qrh = qr[rows, j * Dr:(j + 1) * Dr]
            sc = lax.dot_general(qh, kh, (((1,), (1,)), ((), ())),
                                 preferred_element_type=f32)
            sc = sc + lax.dot_general(qrh, krb, (((1,), (1,)), ((), ())),
                                      preferred_element_type=f32)
            sc = sc * SCALE
            m = jnp.max(sc, axis=-1, keepdims=True)
            e = jnp.exp(sc - m)
            p = (e / jnp.sum(e, axis=-1, keepdims=True)).astype(bf16)
            ag_snd[j][rows, :] = jnp.dot(
                p, v_mine[rows, cols], preferred_element_type=f32
            ).astype(bf16)

    def wo_half(i, half):
        return wo_ref[pl.ds(i * KC + half * Dh, Dh), :]

    ag_r = [rdma(ag_snd_r, ag_rcv_r.at[h], ag_ssem_r.at[h],
                 ag_rsem_r.at[h], right) for h in range(N_DEV - 1)]
    ag_l = [rdma(ag_snd_l, ag_rcv_l.at[h], ag_ssem_l.at[h],
                 ag_rsem_l.at[h], left) for h in range(N_DEV - 1)]
    ag_r[0].start()
    ag_l[0].start()
    out_ref[...] = jnp.dot(ag_snd_r[...], wo_half(my, 0),
                           preferred_element_type=f32)
    out_ref[...] = out_ref[...] + jnp.dot(ag_snd_l[...], wo_half(my, 1),
                                          preferred_element_type=f32)
    for h in range(N_DEV - 1):
        ag_r[h].wait()
        ag_l[h].wait()
        if h < N_DEV - 2:
            ag_snd_r[...] = ag_rcv_r[h, :, :]
            ag_snd_l[...] = ag_rcv_l[h, :, :]
            ag_r[h + 1].start()
            ag_l[h + 1].start()
        r = pos(2 * N_DEV - 1 - h)
        l = pos(1 + h)
        out_ref[...] = out_ref[...] + jnp.dot(
            ag_rcv_r[h, :, :], wo_half(r, 0), preferred_element_type=f32)
        out_ref[...] = out_ref[...] + jnp.dot(
            ag_rcv_l[h, :, :], wo_half(l, 1), preferred_element_type=f32)


def kernel(x, Wdkv, Wuk, Wuv, Wq, Wqr, Wkr, Wo):
    idx = lax.axis_index("i")
    xf = x.reshape(BS, D).astype(bf16)
    wq_loc = lax.dynamic_slice(Wq, (0, idx * KC), (D, KC)).astype(bf16)
    wqr_loc = lax.dynamic_slice(Wqr, (0, idx * QRC), (D, QRC)).astype(bf16)

    out = pl.pallas_call(
        _body,
        out_shape=jax.ShapeDtypeStruct((BS, D), jnp.float32),
        in_specs=[pl.BlockSpec(memory_space=pltpu.VMEM)] * 8,
        out_specs=pl.BlockSpec(memory_space=pltpu.VMEM),
        scratch_shapes=[
            pltpu.VMEM((BS, KC), bf16),
            pltpu.VMEM((BS, KC), bf16),
            pltpu.VMEM((N_DEV - 1, BS, KC), bf16),
            pltpu.VMEM((N_DEV - 1, BS, KC), bf16),
            pltpu.VMEM((BS, Dh), bf16),
            pltpu.VMEM((BS, Dh), bf16),
            pltpu.VMEM((N_DEV - 1, BS, Dh), bf16),
            pltpu.VMEM((N_DEV - 1, BS, Dh), bf16),
            pltpu.SemaphoreType.DMA((N_DEV - 1,)),
            pltpu.SemaphoreType.DMA((N_DEV - 1,)),
            pltpu.SemaphoreType.DMA((N_DEV - 1,)),
            pltpu.SemaphoreType.DMA((N_DEV - 1,)),
            pltpu.SemaphoreType.DMA((N_DEV - 1,)),
            pltpu.SemaphoreType.DMA((N_DEV - 1,)),
            pltpu.SemaphoreType.DMA((N_DEV - 1,)),
            pltpu.SemaphoreType.DMA((N_DEV - 1,)),
        ],
        compiler_params=pltpu.CompilerParams(
            vmem_limit_bytes=62 * 1024 * 1024,
        ),
    )(xf, Wdkv.astype(bf16), Wuk.astype(bf16), Wuv.astype(bf16),
      wq_loc, wqr_loc, Wkr.astype(bf16), Wo.astype(bf16))
    return out.reshape(B, S, D)
